# baseline (speedup 1.0000x reference)
import dataclasses
import functools

import jax
import jax.numpy as jnp
from jax import lax
from jax.experimental import pallas as pl
from jax.experimental.pallas import tpu as pltpu
from jax.experimental.pallas import tpu_sc as plsc

N = 16384
D = 64
NC = 2
NS = 16
NW = NC * NS
Q = 128
NQ = 2
SB = Q * NQ
ROWS_PER_TILE = N // NS
NGB = 3
NIB = 4


def _sc_spmm(rows, cols, vals, weight):
    nnz = rows.shape[0]
    nsb = pl.cdiv(nnz, NW * SB)
    epw = nsb * SB
    pad = epw * NW - nnz
    if pad:
        rows = jnp.concatenate([rows, jnp.zeros((pad,), rows.dtype)])
        cols = jnp.concatenate([cols, jnp.zeros((pad,), cols.dtype)])
        vals = jnp.concatenate([vals, jnp.zeros((pad,), vals.dtype)])
    rows = rows.reshape(-1, Q)

    mesh = plsc.VectorSubcoreMesh(core_axis_name="c", subcore_axis_name="s")
    cp = pltpu.CompilerParams()
    if "needs_layout_passes" in pltpu.CompilerParams.__dataclass_fields__:
        cp = dataclasses.replace(cp, needs_layout_passes=False)
    if "use_tc_tiling_on_sc" in pltpu.CompilerParams.__dataclass_fields__:
        cp = dataclasses.replace(cp, use_tc_tiling_on_sc=False)

    @functools.partial(
        pl.kernel,
        mesh=mesh,
        compiler_params=cp,
        out_type=jax.ShapeDtypeStruct((NC, N, D), jnp.float32),
        scratch_types=[
            pltpu.VMEM((NIB, NQ, Q), jnp.int32),
            pltpu.VMEM((NIB, SB), jnp.int32),
            pltpu.VMEM((NIB, SB), jnp.float32),
            pltpu.VMEM((NGB, SB, D), jnp.float32),
            pltpu.VMEM_SHARED((N, D), jnp.float32),
            pltpu.SemaphoreType.DMA((NIB,)),
            pltpu.SemaphoreType.DMA((NGB,)),
            pltpu.SemaphoreType.DMA((NGB,)),
        ],
    )
    def spmm(rows_hbm, cols_hbm, vals_hbm, w_hbm, part_hbm,
             rows_v, cols_v, vals_v, g_v, acc, sem_i, sem_g, sem_s):
        cid = lax.axis_index("c")
        sid = lax.axis_index("s")

        @pl.loop(0, SB)
        def _(r):
            for j in range(D // 16):
                g_v[0, r, pl.ds(j * 16, 16)] = jnp.zeros((16,), jnp.float32)

        zbase = sid * ROWS_PER_TILE
        for z in range(ROWS_PER_TILE // SB):
            pltpu.sync_copy(g_v.at[0], acc.at[pl.ds(zbase + z * SB, SB)])
        plsc.subcore_barrier()

        wid = sid * NC + cid
        base = wid * epw

        def issue_idx(s):
            b = s % NIB
            off = base + s * SB
            pltpu.async_copy(rows_hbm.at[pl.ds(off // Q, NQ)], rows_v.at[b],
                             sem_i.at[b])
            pltpu.async_copy(cols_hbm.at[pl.ds(off, SB)], cols_v.at[b],
                             sem_i.at[b])
            pltpu.async_copy(vals_hbm.at[pl.ds(off, SB)], vals_v.at[b],
                             sem_i.at[b])

        def wait_idx(s):
            b = s % NIB
            pltpu.make_async_copy(rows_hbm.at[pl.ds(0, NQ)], rows_v.at[b],
                                  sem_i.at[b]).wait()
            pltpu.make_async_copy(cols_hbm.at[pl.ds(0, SB)], cols_v.at[b],
                                  sem_i.at[b]).wait()
            pltpu.make_async_copy(vals_hbm.at[pl.ds(0, SB)], vals_v.at[b],
                                  sem_i.at[b]).wait()

        def issue_gather(s):
            b = s % NGB
            pltpu.async_copy(w_hbm.at[cols_v.at[s % NIB]], g_v.at[b],
                             sem_g.at[b])

        def wait_gather(s):
            b = s % NGB
            pltpu.make_async_copy(w_hbm.at[pl.ds(0, SB)], g_v.at[b],
                                  sem_g.at[b]).wait()

        def issue_scatter(s):
            b3, b4 = s % NGB, s % NIB
            for q in range(NQ):
                pltpu.async_copy(g_v.at[b3, pl.ds(q * Q, Q)],
                                 acc.at[rows_v.at[b4, q]], sem_s.at[b3],
                                 add=True)

        def wait_scatter(s):
            b = s % NGB
            pltpu.make_async_copy(g_v.at[b], acc.at[pl.ds(0, SB)],
                                  sem_s.at[b]).wait()

        issue_idx(0)
        wait_idx(0)
        issue_gather(0)
        if nsb > 1:
            issue_idx(1)

        @pl.loop(0, nsb)
        def _(s):
            b3 = s % NGB
            b4 = s % NIB

            @pl.when(s >= 2)
            def _():
                wait_scatter(s - 2)

            @pl.when(s + 1 < nsb)
            def _():
                wait_idx(s + 1)
                issue_gather(s + 1)

            wait_gather(s)

            dnums = lax.GatherDimensionNumbers(
                offset_dims=(), collapsed_slice_dims=(0,),
                start_index_map=(0,))

            @pl.loop(0, SB, step=16)
            def _(e0):
                v = vals_v[b4, pl.ds(e0, 16)]
                for u in range(16):
                    vb = lax.gather(
                        v, jnp.full((16, 1), u, jnp.int32), dnums, (1,),
                        mode=lax.GatherScatterMode.PROMISE_IN_BOUNDS)
                    for j in range(D // 16):
                        g_v[b3, e0 + u, pl.ds(j * 16, 16)] = (
                            g_v[b3, e0 + u, pl.ds(j * 16, 16)] * vb)

            issue_scatter(s)

            @pl.when(s + 2 < nsb)
            def _():
                issue_idx(s + 2)

        for t in range(max(nsb - 2, 0), nsb):
            wait_scatter(t)

        plsc.subcore_barrier()
        pltpu.sync_copy(
            acc.at[pl.ds(sid * ROWS_PER_TILE, ROWS_PER_TILE)],
            part_hbm.at[cid, pl.ds(sid * ROWS_PER_TILE, ROWS_PER_TILE)],
        )

    return spmm(rows, cols, vals, weight)


def _tc_combine(part):
    def body(p_ref, o_ref):
        o_ref[...] = p_ref[0] + p_ref[1]

    BR = 512
    return pl.pallas_call(
        body,
        out_shape=jax.ShapeDtypeStruct((N, D), jnp.float32),
        grid=(N // BR,),
        in_specs=[pl.BlockSpec((NC, BR, D), lambda i: (0, i, 0))],
        out_specs=pl.BlockSpec((BR, D), lambda i: (i, 0)),
    )(part)


def kernel(rows, cols, vals, weight):
    rows = rows.astype(jnp.int32)
    cols = cols.astype(jnp.int32)
    part = _sc_spmm(rows, cols, vals, weight)
    return _tc_combine(part)
